# baseline (speedup 1.0000x reference)
import functools

import jax
import jax.numpy as jnp
from jax import lax
from jax.experimental import pallas as pl
from jax.experimental.pallas import tpu as pltpu
from jax.experimental.pallas import tpu_sc as plsc

NNODE = 10000
D = 128
NGRP = 64
BNEPS = 1e-5
NC, NS, L = 2, 16, 16
NW = NC * NS
C = 128
CH = 80
EPAD = NW * CH * C
TR = 632
NPT = NS * TR
SB = 16
NSTG = CH // SB

_mesh = plsc.VectorSubcoreMesh(core_axis_name="c", subcore_axis_name="s")


def _make_seg_rows_body(with_deg):
    def body_fn(*refs):
        if with_deg:
            (a_hbm, src_hbm, dst_hbm, z2_hbm, z1_hbm, s_out, d_out,
             srcv, dstv, rows0, rows1, ones_v, tmp1, acc, dacc,
             sem0, sem1, semS0, semS1, semD) = refs
        else:
            (a_hbm, src_hbm, dst_hbm, z2_hbm, s_out,
             srcv, dstv, rows0, rows1, acc,
             sem0, sem1, semS0, semS1) = refs
        c = lax.axis_index("c")
        s = lax.axis_index("s")
        wid = c * NS + s
        pltpu.sync_copy(z2_hbm, acc.at[pl.ds(s * TR, TR)])
        if with_deg:
            pltpu.sync_copy(z1_hbm, tmp1)
            pltpu.sync_copy(tmp1, dacc.at[pl.ds(s * TR, TR)])
            for k in range(C // L):
                ones_v[pl.ds(k * L, L)] = jnp.ones((L,), jnp.float32)
        plsc.subcore_barrier()

        def wait_bytes_of(src, dst, sem):
            pltpu.make_async_copy(src, dst, sem).wait()

        def scat(j, buf):
            pltpu.sync_copy(buf, acc.at[dstv.at[j]], add=True)
            if with_deg:
                pltpu.async_copy(ones_v, dacc.at[dstv.at[j]], semD, add=True)

        for g in range(NSTG):
            base = wid * CH + g * SB
            pltpu.sync_copy(src_hbm.at[pl.ds(base, SB)], srcv)
            pltpu.sync_copy(dst_hbm.at[pl.ds(base, SB)], dstv)
            def agather(j, buf, sem):
                hc = C // 2
                pltpu.async_copy(a_hbm.at[srcv.at[j, pl.ds(0, hc)]],
                                 buf.at[pl.ds(0, hc)], sem)
                pltpu.async_copy(a_hbm.at[srcv.at[j, pl.ds(hc, hc)]],
                                 buf.at[pl.ds(hc, hc)], sem)

            agather(0, rows0, sem0)

            def body2(i, carry):
                j0 = i * 2
                agather(j0 + 1, rows1, sem1)
                wait_bytes_of(a_hbm.at[srcv.at[0]], rows0, sem0)
                scat(j0, rows0)
                agather(j0 + 2, rows0, sem0)
                wait_bytes_of(a_hbm.at[srcv.at[0]], rows1, sem1)
                scat(j0 + 1, rows1)
                return carry

            lax.fori_loop(0, SB // 2 - 1, body2, 0)
            agather(SB - 1, rows1, sem1)
            wait_bytes_of(a_hbm.at[srcv.at[0]], rows0, sem0)
            scat(SB - 2, rows0)
            wait_bytes_of(a_hbm.at[srcv.at[0]], rows1, sem1)
            scat(SB - 1, rows1)
            if with_deg:
                wait_bytes_of(src_hbm.at[pl.ds(0, SB)], srcv, semD)

        plsc.subcore_barrier()
        pltpu.sync_copy(acc.at[pl.ds(s * TR, TR)],
                        s_out.at[c, pl.ds(s * TR, TR)])
        if with_deg:
            pltpu.sync_copy(dacc.at[pl.ds(s * TR, TR)], tmp1)
            pltpu.sync_copy(tmp1, d_out.at[pl.ds(c * NPT + s * TR, TR)])
    return body_fn


_sc_seg_rows = pl.kernel(
    _make_seg_rows_body(True),
    out_type=(
        jax.ShapeDtypeStruct((NC, NPT, D), jnp.float32),
        jax.ShapeDtypeStruct((NC * NPT,), jnp.float32),
    ),
    mesh=_mesh,
    scratch_types=[
        pltpu.VMEM((SB, C), jnp.int32),
        pltpu.VMEM((SB, C), jnp.int32),
        pltpu.VMEM((C, D), jnp.float32),
        pltpu.VMEM((C, D), jnp.float32),
        pltpu.VMEM((C,), jnp.float32),
        pltpu.VMEM((TR,), jnp.float32),
        pltpu.VMEM_SHARED((NPT, D), jnp.float32),
        pltpu.VMEM_SHARED((NPT,), jnp.float32),
        pltpu.SemaphoreType.DMA,
        pltpu.SemaphoreType.DMA,
        pltpu.SemaphoreType.DMA,
        pltpu.SemaphoreType.DMA,
        pltpu.SemaphoreType.DMA,
    ],
)


@functools.partial(
    pl.kernel,
    out_type=jax.ShapeDtypeStruct((NC * NPT,), jnp.float32),
    mesh=_mesh,
    scratch_types=[
        pltpu.VMEM((CH, C), jnp.int32),
        pltpu.VMEM((CH, C), jnp.int32),
        pltpu.VMEM((C,), jnp.float32),
        pltpu.VMEM((TR,), jnp.float32),
        pltpu.VMEM_SHARED((NPT,), jnp.float32),
    ],
)
def _sc_seg_scal(a_hbm, src_hbm, dst_hbm, z1_hbm, s_out,
                 srcv, dstv, vals, tmp1, acc):
    c = lax.axis_index("c")
    s = lax.axis_index("s")
    wid = c * NS + s
    pltpu.sync_copy(z1_hbm, tmp1)
    pltpu.sync_copy(tmp1, acc.at[pl.ds(s * TR, TR)])
    pltpu.sync_copy(src_hbm.at[pl.ds(wid * CH, CH)], srcv)
    pltpu.sync_copy(dst_hbm.at[pl.ds(wid * CH, CH)], dstv)
    plsc.subcore_barrier()

    def body(j, carry):
        pltpu.sync_copy(a_hbm.at[srcv.at[j]], vals)
        pltpu.sync_copy(vals, acc.at[dstv.at[j]], add=True)
        return carry

    lax.fori_loop(0, CH, body, 0)

    plsc.subcore_barrier()
    pltpu.sync_copy(acc.at[pl.ds(s * TR, TR)], tmp1)
    pltpu.sync_copy(tmp1, s_out.at[pl.ds(c * NPT + s * TR, TR)])


def _dotT(x, w):
    return lax.dot_general(x, w, (((1,), (1,)), ((), ())),
                           preferred_element_type=jnp.float32)


def _tc_pre_body(x_ref, wl_ref, wr_ref, bl_ref, a_ref, r_ref):
    x = x_ref[...]
    a_ref[...] = _dotT(x, wl_ref[...])
    r_ref[...] = _dotT(x, wr_ref[...]) + bl_ref[...]


_tc_pre = pl.pallas_call(
    _tc_pre_body,
    out_shape=(jax.ShapeDtypeStruct((NPT, D), jnp.float32),
               jax.ShapeDtypeStruct((NPT, D), jnp.float32)),
)


def _bn_relu(y, g, b, rm, rv):
    return jnp.maximum((y - rm) * lax.rsqrt(rv + BNEPS) * g + b, 0.0)


def _combine(sp, dp, r):
    s = sp[0] + sp[1]
    deg = jnp.maximum(dp[0] + dp[1], 1.0)
    return s / deg[:, None] + r


def _tc_mid_body(sp_ref, dp_ref, r1_ref, g_ref, b_ref, rm_ref, rv_ref,
                 wl_ref, wr_ref, bl_ref, a_ref, r_ref):
    y = _combine(sp_ref[...], dp_ref[...], r1_ref[...])
    h = _bn_relu(y, g_ref[...], b_ref[...], rm_ref[...], rv_ref[...])
    a_ref[...] = _dotT(h, wl_ref[...])
    r_ref[...] = _dotT(h, wr_ref[...]) + bl_ref[...]


_tc_mid = pl.pallas_call(
    _tc_mid_body,
    out_shape=(jax.ShapeDtypeStruct((NPT, D), jnp.float32),
               jax.ShapeDtypeStruct((NPT, D), jnp.float32)),
)


def _tc_fin_body(sp_ref, dp_ref, r2_ref, g_ref, b_ref, rm_ref, rv_ref,
                 wl_ref, wr_ref, a_ref, r_ref):
    y = _combine(sp_ref[...], dp_ref[...], r2_ref[...])
    h = _bn_relu(y, g_ref[...], b_ref[...], rm_ref[...], rv_ref[...])
    a_ref[...] = jnp.sum(h * wl_ref[...], axis=1, keepdims=True)
    r_ref[...] = jnp.sum(h * wr_ref[...], axis=1, keepdims=True)


_tc_fin = pl.pallas_call(
    _tc_fin_body,
    out_shape=(jax.ShapeDtypeStruct((NPT, 1), jnp.float32),
               jax.ShapeDtypeStruct((NPT, 1), jnp.float32)),
)


def _tc_pool_body(s3_ref, dp_ref, r3_ref, bt_ref, b3_ref, out_ref):
    s3 = s3_ref[0] + s3_ref[1]
    deg = jnp.maximum(dp_ref[0] + dp_ref[1], 1.0)
    h3 = (s3 / deg)[:, None] + r3_ref[...] + b3_ref[...]
    gid = lax.broadcasted_iota(jnp.int32, (NGRP, NPT), 0)
    oh = (bt_ref[...] == gid).astype(jnp.float32)
    pool = lax.dot_general(oh, h3, (((1,), (0,)), ((), ())),
                           preferred_element_type=jnp.float32)
    cnt = jnp.sum(oh, axis=1, keepdims=True)
    out_ref[...] = pool / jnp.maximum(cnt, 1.0)


_tc_pool = pl.pallas_call(
    _tc_pool_body,
    out_shape=jax.ShapeDtypeStruct((NGRP, 1), jnp.float32),
)


def kernel(x, edge_index, batch, W1l, b1l, W1r, bn1_g, bn1_b, bn1_rm, bn1_rv,
           W2l, b2l, W2r, bn2_g, bn2_b, bn2_rm, bn2_rv, W3l, b3l, W3r):
    src = edge_index[0]
    dst = edge_index[1]
    e = src.shape[0]
    srcp = jnp.concatenate(
        [src, jnp.zeros((EPAD - e,), jnp.int32)]).reshape(NW * CH, C)
    dstp = jnp.concatenate(
        [dst, jnp.full((EPAD - e,), NNODE, jnp.int32)]).reshape(NW * CH, C)
    xp = jnp.zeros((NPT, D), jnp.float32).at[:NNODE].set(x)
    z2 = jnp.zeros((TR, D), jnp.float32)
    z1 = jnp.zeros((TR,), jnp.float32)
    bt = jnp.full((1, NPT), -1, jnp.int32).at[0, :NNODE].set(batch)

    r = lambda v: v.reshape(1, -1)
    A1, R1 = _tc_pre(xp, W1l, W1r, r(b1l))
    S1, degf = _sc_seg_rows(A1, srcp, dstp, z2, z1)
    degp = degf.reshape(NC, NPT)
    A2, R2 = _tc_mid(S1, degp, R1, r(bn1_g), r(bn1_b), r(bn1_rm), r(bn1_rv),
                     W2l, W2r, r(b2l))
    S2, _ = _sc_seg_rows(A2, srcp, dstp, z2, z1)
    a3, r3 = _tc_fin(S2, degp, R2, r(bn2_g), r(bn2_b), r(bn2_rm), r(bn2_rv),
                     W3l, W3r)
    s3p = _sc_seg_scal(a3.reshape(NPT), srcp, dstp, z1).reshape(NC, NPT)
    return _tc_pool(s3p, degp, r3, bt, b3l.reshape(1, 1))


# speedup vs baseline: 1.0226x; 1.0226x over previous
import functools

import jax
import jax.numpy as jnp
from jax import lax
from jax.experimental import pallas as pl
from jax.experimental.pallas import tpu as pltpu
from jax.experimental.pallas import tpu_sc as plsc

NNODE = 10000
D = 128
NGRP = 64
BNEPS = 1e-5
NC, NS, L = 2, 16, 16
NW = NC * NS
C = 128
CH = 80
EPAD = NW * CH * C
TR = 632
NPT = NS * TR
SB = 16
NSTG = CH // SB

_mesh = plsc.VectorSubcoreMesh(core_axis_name="c", subcore_axis_name="s")


def _make_seg_rows_body(with_deg):
    def body_fn(*refs):
        if with_deg:
            (a_hbm, src_hbm, dst_hbm, z2_hbm, z1_hbm, s_out, d_out,
             srcv, dstv, rows0, rows1, ones_v, tmp1, acc, dacc,
             sem0, sem1, semS0, semS1, semD) = refs
        else:
            (a_hbm, src_hbm, dst_hbm, z2_hbm, s_out,
             srcv, dstv, rows0, rows1, acc,
             sem0, sem1, semS0, semS1) = refs
        c = lax.axis_index("c")
        s = lax.axis_index("s")
        wid = c * NS + s
        pltpu.sync_copy(z2_hbm, acc.at[pl.ds(s * TR, TR)])
        if with_deg:
            pltpu.sync_copy(z1_hbm, tmp1)
            pltpu.sync_copy(tmp1, dacc.at[pl.ds(s * TR, TR)])
            for k in range(C // L):
                ones_v[pl.ds(k * L, L)] = jnp.ones((L,), jnp.float32)
        plsc.subcore_barrier()

        def wait_bytes_of(src, dst, sem):
            pltpu.make_async_copy(src, dst, sem).wait()

        def scat(j, buf):
            pltpu.sync_copy(buf, acc.at[dstv.at[j]], add=True)
            if with_deg:
                pltpu.async_copy(ones_v, dacc.at[dstv.at[j]], semD, add=True)

        for g in range(NSTG):
            base = wid * CH + g * SB
            pltpu.sync_copy(src_hbm.at[pl.ds(base, SB)], srcv)
            pltpu.sync_copy(dst_hbm.at[pl.ds(base, SB)], dstv)
            pltpu.async_copy(a_hbm.at[srcv.at[0]], rows0, sem0)

            def body2(i, carry):
                j0 = i * 2
                pltpu.async_copy(a_hbm.at[srcv.at[j0 + 1]], rows1, sem1)
                wait_bytes_of(a_hbm.at[srcv.at[0]], rows0, sem0)
                scat(j0, rows0)
                pltpu.async_copy(a_hbm.at[srcv.at[j0 + 2]], rows0, sem0)
                wait_bytes_of(a_hbm.at[srcv.at[0]], rows1, sem1)
                scat(j0 + 1, rows1)
                return carry

            lax.fori_loop(0, SB // 2 - 1, body2, 0)
            pltpu.async_copy(a_hbm.at[srcv.at[SB - 1]], rows1, sem1)
            wait_bytes_of(a_hbm.at[srcv.at[0]], rows0, sem0)
            scat(SB - 2, rows0)
            wait_bytes_of(a_hbm.at[srcv.at[0]], rows1, sem1)
            scat(SB - 1, rows1)
            if with_deg:
                wait_bytes_of(src_hbm.at[pl.ds(0, SB)], srcv, semD)

        plsc.subcore_barrier()
        pltpu.sync_copy(acc.at[pl.ds(s * TR, TR)],
                        s_out.at[c, pl.ds(s * TR, TR)])
        if with_deg:
            pltpu.sync_copy(dacc.at[pl.ds(s * TR, TR)], tmp1)
            pltpu.sync_copy(tmp1, d_out.at[pl.ds(c * NPT + s * TR, TR)])
    return body_fn


_sc_seg_rows = pl.kernel(
    _make_seg_rows_body(True),
    out_type=(
        jax.ShapeDtypeStruct((NC, NPT, D), jnp.float32),
        jax.ShapeDtypeStruct((NC * NPT,), jnp.float32),
    ),
    mesh=_mesh,
    scratch_types=[
        pltpu.VMEM((SB, C), jnp.int32),
        pltpu.VMEM((SB, C), jnp.int32),
        pltpu.VMEM((C, D), jnp.float32),
        pltpu.VMEM((C, D), jnp.float32),
        pltpu.VMEM((C,), jnp.float32),
        pltpu.VMEM((TR,), jnp.float32),
        pltpu.VMEM_SHARED((NPT, D), jnp.float32),
        pltpu.VMEM_SHARED((NPT,), jnp.float32),
        pltpu.SemaphoreType.DMA,
        pltpu.SemaphoreType.DMA,
        pltpu.SemaphoreType.DMA,
        pltpu.SemaphoreType.DMA,
        pltpu.SemaphoreType.DMA,
    ],
)


@functools.partial(
    pl.kernel,
    out_type=jax.ShapeDtypeStruct((NC * NPT,), jnp.float32),
    mesh=_mesh,
    scratch_types=[
        pltpu.VMEM((CH, C), jnp.int32),
        pltpu.VMEM((CH, C), jnp.int32),
        pltpu.VMEM((C,), jnp.float32),
        pltpu.VMEM((C,), jnp.float32),
        pltpu.VMEM((TR,), jnp.float32),
        pltpu.VMEM_SHARED((NPT,), jnp.float32),
        pltpu.SemaphoreType.DMA,
        pltpu.SemaphoreType.DMA,
    ],
)
def _sc_seg_scal(a_hbm, src_hbm, dst_hbm, z1_hbm, s_out,
                 srcv, dstv, vals, vals1, tmp1, acc, sem0, sem1):
    c = lax.axis_index("c")
    s = lax.axis_index("s")
    wid = c * NS + s
    pltpu.sync_copy(z1_hbm, tmp1)
    pltpu.sync_copy(tmp1, acc.at[pl.ds(s * TR, TR)])
    pltpu.sync_copy(src_hbm.at[pl.ds(wid * CH, CH)], srcv)
    pltpu.sync_copy(dst_hbm.at[pl.ds(wid * CH, CH)], dstv)
    plsc.subcore_barrier()

    def wait_v(buf, sem):
        pltpu.make_async_copy(a_hbm.at[srcv.at[0]], buf, sem).wait()

    pltpu.async_copy(a_hbm.at[srcv.at[0]], vals, sem0)

    def body(i, carry):
        j0 = i * 2
        pltpu.async_copy(a_hbm.at[srcv.at[j0 + 1]], vals1, sem1)
        wait_v(vals, sem0)
        pltpu.sync_copy(vals, acc.at[dstv.at[j0]], add=True)
        pltpu.async_copy(a_hbm.at[srcv.at[j0 + 2]], vals, sem0)
        wait_v(vals1, sem1)
        pltpu.sync_copy(vals1, acc.at[dstv.at[j0 + 1]], add=True)
        return carry

    lax.fori_loop(0, CH // 2 - 1, body, 0)
    pltpu.async_copy(a_hbm.at[srcv.at[CH - 1]], vals1, sem1)
    wait_v(vals, sem0)
    pltpu.sync_copy(vals, acc.at[dstv.at[CH - 2]], add=True)
    wait_v(vals1, sem1)
    pltpu.sync_copy(vals1, acc.at[dstv.at[CH - 1]], add=True)

    plsc.subcore_barrier()
    pltpu.sync_copy(acc.at[pl.ds(s * TR, TR)], tmp1)
    pltpu.sync_copy(tmp1, s_out.at[pl.ds(c * NPT + s * TR, TR)])


def _dotT(x, w):
    return lax.dot_general(x, w, (((1,), (1,)), ((), ())),
                           preferred_element_type=jnp.float32)


def _tc_pre_body(x_ref, wl_ref, wr_ref, bl_ref, a_ref, r_ref):
    x = x_ref[...]
    a_ref[...] = _dotT(x, wl_ref[...])
    r_ref[...] = _dotT(x, wr_ref[...]) + bl_ref[...]


_tc_pre = pl.pallas_call(
    _tc_pre_body,
    out_shape=(jax.ShapeDtypeStruct((NPT, D), jnp.float32),
               jax.ShapeDtypeStruct((NPT, D), jnp.float32)),
)


def _bn_relu(y, g, b, rm, rv):
    return jnp.maximum((y - rm) * lax.rsqrt(rv + BNEPS) * g + b, 0.0)


def _combine(sp, dp, r):
    s = sp[0] + sp[1]
    deg = jnp.maximum(dp[0] + dp[1], 1.0)
    return s / deg[:, None] + r


def _tc_mid_body(sp_ref, dp_ref, r1_ref, g_ref, b_ref, rm_ref, rv_ref,
                 wl_ref, wr_ref, bl_ref, a_ref, r_ref):
    y = _combine(sp_ref[...], dp_ref[...], r1_ref[...])
    h = _bn_relu(y, g_ref[...], b_ref[...], rm_ref[...], rv_ref[...])
    a_ref[...] = _dotT(h, wl_ref[...])
    r_ref[...] = _dotT(h, wr_ref[...]) + bl_ref[...]


_tc_mid = pl.pallas_call(
    _tc_mid_body,
    out_shape=(jax.ShapeDtypeStruct((NPT, D), jnp.float32),
               jax.ShapeDtypeStruct((NPT, D), jnp.float32)),
)


def _tc_fin_body(sp_ref, dp_ref, r2_ref, g_ref, b_ref, rm_ref, rv_ref,
                 wl_ref, wr_ref, a_ref, r_ref):
    y = _combine(sp_ref[...], dp_ref[...], r2_ref[...])
    h = _bn_relu(y, g_ref[...], b_ref[...], rm_ref[...], rv_ref[...])
    a_ref[...] = jnp.sum(h * wl_ref[...], axis=1, keepdims=True)
    r_ref[...] = jnp.sum(h * wr_ref[...], axis=1, keepdims=True)


_tc_fin = pl.pallas_call(
    _tc_fin_body,
    out_shape=(jax.ShapeDtypeStruct((NPT, 1), jnp.float32),
               jax.ShapeDtypeStruct((NPT, 1), jnp.float32)),
)


def _tc_pool_body(s3_ref, dp_ref, r3_ref, bt_ref, b3_ref, out_ref):
    s3 = s3_ref[0] + s3_ref[1]
    deg = jnp.maximum(dp_ref[0] + dp_ref[1], 1.0)
    h3 = (s3 / deg)[:, None] + r3_ref[...] + b3_ref[...]
    gid = lax.broadcasted_iota(jnp.int32, (NGRP, NPT), 0)
    oh = (bt_ref[...] == gid).astype(jnp.float32)
    pool = lax.dot_general(oh, h3, (((1,), (0,)), ((), ())),
                           preferred_element_type=jnp.float32)
    cnt = jnp.sum(oh, axis=1, keepdims=True)
    out_ref[...] = pool / jnp.maximum(cnt, 1.0)


_tc_pool = pl.pallas_call(
    _tc_pool_body,
    out_shape=jax.ShapeDtypeStruct((NGRP, 1), jnp.float32),
)


def kernel(x, edge_index, batch, W1l, b1l, W1r, bn1_g, bn1_b, bn1_rm, bn1_rv,
           W2l, b2l, W2r, bn2_g, bn2_b, bn2_rm, bn2_rv, W3l, b3l, W3r):
    src = edge_index[0]
    dst = edge_index[1]
    e = src.shape[0]
    srcp = jnp.concatenate(
        [src, jnp.zeros((EPAD - e,), jnp.int32)]).reshape(NW * CH, C)
    dstp = jnp.concatenate(
        [dst, jnp.full((EPAD - e,), NNODE, jnp.int32)]).reshape(NW * CH, C)
    xp = jnp.zeros((NPT, D), jnp.float32).at[:NNODE].set(x)
    z2 = jnp.zeros((TR, D), jnp.float32)
    z1 = jnp.zeros((TR,), jnp.float32)
    bt = jnp.full((1, NPT), -1, jnp.int32).at[0, :NNODE].set(batch)

    r = lambda v: v.reshape(1, -1)
    A1, R1 = _tc_pre(xp, W1l, W1r, r(b1l))
    S1, degf = _sc_seg_rows(A1, srcp, dstp, z2, z1)
    degp = degf.reshape(NC, NPT)
    A2, R2 = _tc_mid(S1, degp, R1, r(bn1_g), r(bn1_b), r(bn1_rm), r(bn1_rv),
                     W2l, W2r, r(b2l))
    S2, _ = _sc_seg_rows(A2, srcp, dstp, z2, z1)
    a3, r3 = _tc_fin(S2, degp, R2, r(bn2_g), r(bn2_b), r(bn2_rm), r(bn2_rv),
                     W3l, W3r)
    s3p = _sc_seg_scal(a3.reshape(NPT), srcp, dstp, z1).reshape(NC, NPT)
    return _tc_pool(s3p, degp, r3, bt, b3l.reshape(1, 1))


# speedup vs baseline: 1.0387x; 1.0157x over previous
import functools

import jax
import jax.numpy as jnp
from jax import lax
from jax.experimental import pallas as pl
from jax.experimental.pallas import tpu as pltpu
from jax.experimental.pallas import tpu_sc as plsc

NNODE = 10000
D = 128
NGRP = 64
BNEPS = 1e-5
NC, NS, L = 2, 16, 16
NW = NC * NS
C = 128
CH = 80
EPAD = NW * CH * C
TR = 632
NPT = NS * TR
SB = 40
NSTG = CH // SB

_mesh = plsc.VectorSubcoreMesh(core_axis_name="c", subcore_axis_name="s")


def _make_seg_rows_body(with_deg):
    def body_fn(*refs):
        if with_deg:
            (a_hbm, src_hbm, dst_hbm, z2_hbm, z1_hbm, s_out, d_out,
             srcv, dstv, rows0, rows1, ones_v, tmp1, acc, dacc,
             sem0, sem1, semS0, semS1, semD) = refs
        else:
            (a_hbm, src_hbm, dst_hbm, z2_hbm, s_out,
             srcv, dstv, rows0, rows1, acc,
             sem0, sem1, semS0, semS1) = refs
        c = lax.axis_index("c")
        s = lax.axis_index("s")
        wid = c * NS + s
        pltpu.sync_copy(z2_hbm, acc.at[pl.ds(s * TR, TR)])
        if with_deg:
            pltpu.sync_copy(z1_hbm, tmp1)
            pltpu.sync_copy(tmp1, dacc.at[pl.ds(s * TR, TR)])
            for k in range(C // L):
                ones_v[pl.ds(k * L, L)] = jnp.ones((L,), jnp.float32)
        plsc.subcore_barrier()

        def wait_bytes_of(src, dst, sem):
            pltpu.make_async_copy(src, dst, sem).wait()

        def scat(j, buf):
            pltpu.sync_copy(buf, acc.at[dstv.at[j]], add=True)
            if with_deg:
                pltpu.async_copy(ones_v, dacc.at[dstv.at[j]], semD, add=True)

        for g in range(NSTG):
            base = wid * CH + g * SB
            pltpu.sync_copy(src_hbm.at[pl.ds(base, SB)], srcv)
            pltpu.sync_copy(dst_hbm.at[pl.ds(base, SB)], dstv)
            pltpu.async_copy(a_hbm.at[srcv.at[0]], rows0, sem0)

            def body2(i, carry):
                j0 = i * 2
                pltpu.async_copy(a_hbm.at[srcv.at[j0 + 1]], rows1, sem1)
                wait_bytes_of(a_hbm.at[srcv.at[0]], rows0, sem0)
                scat(j0, rows0)
                pltpu.async_copy(a_hbm.at[srcv.at[j0 + 2]], rows0, sem0)
                wait_bytes_of(a_hbm.at[srcv.at[0]], rows1, sem1)
                scat(j0 + 1, rows1)
                return carry

            lax.fori_loop(0, SB // 2 - 1, body2, 0)
            pltpu.async_copy(a_hbm.at[srcv.at[SB - 1]], rows1, sem1)
            wait_bytes_of(a_hbm.at[srcv.at[0]], rows0, sem0)
            scat(SB - 2, rows0)
            wait_bytes_of(a_hbm.at[srcv.at[0]], rows1, sem1)
            scat(SB - 1, rows1)
            if with_deg:
                wait_bytes_of(src_hbm.at[pl.ds(0, SB)], srcv, semD)

        plsc.subcore_barrier()
        pltpu.sync_copy(acc.at[pl.ds(s * TR, TR)],
                        s_out.at[c, pl.ds(s * TR, TR)])
        if with_deg:
            pltpu.sync_copy(dacc.at[pl.ds(s * TR, TR)], tmp1)
            pltpu.sync_copy(tmp1, d_out.at[pl.ds(c * NPT + s * TR, TR)])
    return body_fn


_sc_seg_rows = pl.kernel(
    _make_seg_rows_body(True),
    out_type=(
        jax.ShapeDtypeStruct((NC, NPT, D), jnp.float32),
        jax.ShapeDtypeStruct((NC * NPT,), jnp.float32),
    ),
    mesh=_mesh,
    scratch_types=[
        pltpu.VMEM((SB, C), jnp.int32),
        pltpu.VMEM((SB, C), jnp.int32),
        pltpu.VMEM((C, D), jnp.float32),
        pltpu.VMEM((C, D), jnp.float32),
        pltpu.VMEM((C,), jnp.float32),
        pltpu.VMEM((TR,), jnp.float32),
        pltpu.VMEM_SHARED((NPT, D), jnp.float32),
        pltpu.VMEM_SHARED((NPT,), jnp.float32),
        pltpu.SemaphoreType.DMA,
        pltpu.SemaphoreType.DMA,
        pltpu.SemaphoreType.DMA,
        pltpu.SemaphoreType.DMA,
        pltpu.SemaphoreType.DMA,
    ],
)


@functools.partial(
    pl.kernel,
    out_type=jax.ShapeDtypeStruct((NC * NPT,), jnp.float32),
    mesh=_mesh,
    scratch_types=[
        pltpu.VMEM((CH, C), jnp.int32),
        pltpu.VMEM((CH, C), jnp.int32),
        pltpu.VMEM((C,), jnp.float32),
        pltpu.VMEM((C,), jnp.float32),
        pltpu.VMEM((TR,), jnp.float32),
        pltpu.VMEM_SHARED((NPT,), jnp.float32),
        pltpu.SemaphoreType.DMA,
        pltpu.SemaphoreType.DMA,
    ],
)
def _sc_seg_scal(a_hbm, src_hbm, dst_hbm, z1_hbm, s_out,
                 srcv, dstv, vals, vals1, tmp1, acc, sem0, sem1):
    c = lax.axis_index("c")
    s = lax.axis_index("s")
    wid = c * NS + s
    pltpu.sync_copy(z1_hbm, tmp1)
    pltpu.sync_copy(tmp1, acc.at[pl.ds(s * TR, TR)])
    pltpu.sync_copy(src_hbm.at[pl.ds(wid * CH, CH)], srcv)
    pltpu.sync_copy(dst_hbm.at[pl.ds(wid * CH, CH)], dstv)
    plsc.subcore_barrier()

    def wait_v(buf, sem):
        pltpu.make_async_copy(a_hbm.at[srcv.at[0]], buf, sem).wait()

    pltpu.async_copy(a_hbm.at[srcv.at[0]], vals, sem0)

    def body(i, carry):
        j0 = i * 2
        pltpu.async_copy(a_hbm.at[srcv.at[j0 + 1]], vals1, sem1)
        wait_v(vals, sem0)
        pltpu.sync_copy(vals, acc.at[dstv.at[j0]], add=True)
        pltpu.async_copy(a_hbm.at[srcv.at[j0 + 2]], vals, sem0)
        wait_v(vals1, sem1)
        pltpu.sync_copy(vals1, acc.at[dstv.at[j0 + 1]], add=True)
        return carry

    lax.fori_loop(0, CH // 2 - 1, body, 0)
    pltpu.async_copy(a_hbm.at[srcv.at[CH - 1]], vals1, sem1)
    wait_v(vals, sem0)
    pltpu.sync_copy(vals, acc.at[dstv.at[CH - 2]], add=True)
    wait_v(vals1, sem1)
    pltpu.sync_copy(vals1, acc.at[dstv.at[CH - 1]], add=True)

    plsc.subcore_barrier()
    pltpu.sync_copy(acc.at[pl.ds(s * TR, TR)], tmp1)
    pltpu.sync_copy(tmp1, s_out.at[pl.ds(c * NPT + s * TR, TR)])


def _dotT(x, w):
    return lax.dot_general(x, w, (((1,), (1,)), ((), ())),
                           preferred_element_type=jnp.float32)


def _tc_pre_body(x_ref, wl_ref, wr_ref, bl_ref, a_ref, r_ref):
    x = x_ref[...]
    a_ref[...] = _dotT(x, wl_ref[...])
    r_ref[...] = _dotT(x, wr_ref[...]) + bl_ref[...]


_tc_pre = pl.pallas_call(
    _tc_pre_body,
    out_shape=(jax.ShapeDtypeStruct((NPT, D), jnp.float32),
               jax.ShapeDtypeStruct((NPT, D), jnp.float32)),
)


def _bn_relu(y, g, b, rm, rv):
    return jnp.maximum((y - rm) * lax.rsqrt(rv + BNEPS) * g + b, 0.0)


def _combine(sp, dp, r):
    s = sp[0] + sp[1]
    deg = jnp.maximum(dp[0] + dp[1], 1.0)
    return s / deg[:, None] + r


def _tc_mid_body(sp_ref, dp_ref, r1_ref, g_ref, b_ref, rm_ref, rv_ref,
                 wl_ref, wr_ref, bl_ref, a_ref, r_ref):
    y = _combine(sp_ref[...], dp_ref[...], r1_ref[...])
    h = _bn_relu(y, g_ref[...], b_ref[...], rm_ref[...], rv_ref[...])
    a_ref[...] = _dotT(h, wl_ref[...])
    r_ref[...] = _dotT(h, wr_ref[...]) + bl_ref[...]


_tc_mid = pl.pallas_call(
    _tc_mid_body,
    out_shape=(jax.ShapeDtypeStruct((NPT, D), jnp.float32),
               jax.ShapeDtypeStruct((NPT, D), jnp.float32)),
)


def _tc_fin_body(sp_ref, dp_ref, r2_ref, g_ref, b_ref, rm_ref, rv_ref,
                 wl_ref, wr_ref, a_ref, r_ref):
    y = _combine(sp_ref[...], dp_ref[...], r2_ref[...])
    h = _bn_relu(y, g_ref[...], b_ref[...], rm_ref[...], rv_ref[...])
    a_ref[...] = jnp.sum(h * wl_ref[...], axis=1, keepdims=True)
    r_ref[...] = jnp.sum(h * wr_ref[...], axis=1, keepdims=True)


_tc_fin = pl.pallas_call(
    _tc_fin_body,
    out_shape=(jax.ShapeDtypeStruct((NPT, 1), jnp.float32),
               jax.ShapeDtypeStruct((NPT, 1), jnp.float32)),
)


def _tc_pool_body(s3_ref, dp_ref, r3_ref, bt_ref, b3_ref, out_ref):
    s3 = s3_ref[0] + s3_ref[1]
    deg = jnp.maximum(dp_ref[0] + dp_ref[1], 1.0)
    h3 = (s3 / deg)[:, None] + r3_ref[...] + b3_ref[...]
    gid = lax.broadcasted_iota(jnp.int32, (NGRP, NPT), 0)
    oh = (bt_ref[...] == gid).astype(jnp.float32)
    pool = lax.dot_general(oh, h3, (((1,), (0,)), ((), ())),
                           preferred_element_type=jnp.float32)
    cnt = jnp.sum(oh, axis=1, keepdims=True)
    out_ref[...] = pool / jnp.maximum(cnt, 1.0)


_tc_pool = pl.pallas_call(
    _tc_pool_body,
    out_shape=jax.ShapeDtypeStruct((NGRP, 1), jnp.float32),
)


def kernel(x, edge_index, batch, W1l, b1l, W1r, bn1_g, bn1_b, bn1_rm, bn1_rv,
           W2l, b2l, W2r, bn2_g, bn2_b, bn2_rm, bn2_rv, W3l, b3l, W3r):
    src = edge_index[0]
    dst = edge_index[1]
    e = src.shape[0]
    srcp = jnp.concatenate(
        [src, jnp.zeros((EPAD - e,), jnp.int32)]).reshape(NW * CH, C)
    dstp = jnp.concatenate(
        [dst, jnp.full((EPAD - e,), NNODE, jnp.int32)]).reshape(NW * CH, C)
    xp = jnp.zeros((NPT, D), jnp.float32).at[:NNODE].set(x)
    z2 = jnp.zeros((TR, D), jnp.float32)
    z1 = jnp.zeros((TR,), jnp.float32)
    bt = jnp.full((1, NPT), -1, jnp.int32).at[0, :NNODE].set(batch)

    r = lambda v: v.reshape(1, -1)
    A1, R1 = _tc_pre(xp, W1l, W1r, r(b1l))
    S1, degf = _sc_seg_rows(A1, srcp, dstp, z2, z1)
    degp = degf.reshape(NC, NPT)
    A2, R2 = _tc_mid(S1, degp, R1, r(bn1_g), r(bn1_b), r(bn1_rm), r(bn1_rv),
                     W2l, W2r, r(b2l))
    S2, _ = _sc_seg_rows(A2, srcp, dstp, z2, z1)
    a3, r3 = _tc_fin(S2, degp, R2, r(bn2_g), r(bn2_b), r(bn2_rm), r(bn2_rv),
                     W3l, W3r)
    s3p = _sc_seg_scal(a3.reshape(NPT), srcp, dstp, z1).reshape(NC, NPT)
    return _tc_pool(s3p, degp, r3, bt, b3l.reshape(1, 1))
